# baseline (speedup 1.0000x reference)
import jax
import jax.numpy as jnp
from jax.experimental import pallas as pl
from jax.experimental.pallas import tpu as pltpu


def _round_up(v, m):
    return ((v + m - 1) // m) * m


def _mlp_kernel(x_ref, w1_hbm, b1_hbm, w2_hbm, b2_hbm, o_ref,
                w1s, b1s, w2s, b2s, sems):
    @pl.when(pl.program_id(0) == 0)
    def _load_params():
        pltpu.make_async_copy(w1_hbm, w1s, sems.at[0]).start()
        pltpu.make_async_copy(w2_hbm, w2s, sems.at[1]).start()
        pltpu.make_async_copy(b1_hbm, b1s, sems.at[2]).start()
        pltpu.make_async_copy(b2_hbm, b2s, sems.at[3]).start()
        pltpu.make_async_copy(w1_hbm, w1s, sems.at[0]).wait()
        pltpu.make_async_copy(w2_hbm, w2s, sems.at[1]).wait()
        pltpu.make_async_copy(b1_hbm, b1s, sems.at[2]).wait()
        pltpu.make_async_copy(b2_hbm, b2s, sems.at[3]).wait()

    half = x_ref.shape[0] // 2
    for j in range(2):
        rows = pl.ds(j * half, half)
        h = jnp.dot(x_ref[rows, :], w1s[...],
                    preferred_element_type=jnp.float32)
        h = h + b1s[...]
        h = h * jax.nn.sigmoid(h)
        out = jnp.dot(h, w2s[...], preferred_element_type=jnp.float32)
        o_ref[rows, :] = (out + b2s[...]).astype(o_ref.dtype)


def kernel(w1, b1, w2, b2, x):
    B, Din = x.shape
    D = w1.shape[1]

    Dp = _round_up(D, 128)
    TM = 1024 if B % 2048 == 0 else _round_up(min(512, B), 8)
    Bp = _round_up(B, TM)

    xp = x if Bp == B else jnp.pad(x, ((0, Bp - B), (0, 0)))
    w1p = w1 if Dp == D else jnp.pad(w1, ((0, 0), (0, Dp - D)))
    w2p = w2 if Dp == D else jnp.pad(w2, ((0, Dp - D), (0, Dp - D)))
    b1p = (b1 if Dp == D else jnp.pad(b1, (0, Dp - D))).reshape(1, Dp)
    b2p = (b2 if Dp == D else jnp.pad(b2, (0, Dp - D))).reshape(1, Dp)

    out = pl.pallas_call(
        _mlp_kernel,
        out_shape=jax.ShapeDtypeStruct((Bp, Dp), x.dtype),
        grid=(Bp // TM,),
        in_specs=[
            pl.BlockSpec((TM, Din), lambda i: (i, 0)),
            pl.BlockSpec(memory_space=pl.ANY),
            pl.BlockSpec(memory_space=pl.ANY),
            pl.BlockSpec(memory_space=pl.ANY),
            pl.BlockSpec(memory_space=pl.ANY),
        ],
        out_specs=pl.BlockSpec((TM, Dp), lambda i: (i, 0)),
        scratch_shapes=[
            pltpu.VMEM((Din, Dp), jnp.float32),
            pltpu.VMEM((1, Dp), jnp.float32),
            pltpu.VMEM((Dp, Dp), jnp.float32),
            pltpu.VMEM((1, Dp), jnp.float32),
            pltpu.SemaphoreType.DMA((4,)),
        ],
        compiler_params=pltpu.CompilerParams(
            dimension_semantics=("arbitrary",),
            vmem_limit_bytes=48 * 1024 * 1024,
        ),
    )(xp, w1p, b1p, w2p, b2p)

    return out[:B, :D]


# speedup vs baseline: 1.0146x; 1.0146x over previous
import jax
import jax.numpy as jnp
from jax.experimental import pallas as pl
from jax.experimental.pallas import tpu as pltpu


def _round_up(v, m):
    return ((v + m - 1) // m) * m


def _mlp_kernel(x_ref, w1_ref, b1_ref, w2_ref, b2_ref, o_ref, w1b, w2b):
    @pl.when(pl.program_id(0) == 0)
    def _prep_weights():
        w1b[...] = w1_ref[...].astype(jnp.bfloat16)
        w2b[...] = w2_ref[...].astype(jnp.bfloat16)

    half = x_ref.shape[0] // 2
    for j in range(2):
        rows = pl.ds(j * half, half)
        h = jnp.dot(x_ref[rows, :], w1b[...],
                    preferred_element_type=jnp.float32)
        h = h + b1_ref[...]
        h = h * jax.nn.sigmoid(h)
        out = jnp.dot(h, w2b[...], preferred_element_type=jnp.float32)
        o_ref[rows, :] = (out + b2_ref[...]).astype(o_ref.dtype)


def kernel(w1, b1, w2, b2, x):
    B, Din = x.shape
    D = w1.shape[1]

    Dp = _round_up(D, 128)
    TM = 1024 if B % 2048 == 0 else _round_up(min(512, B), 8)
    Bp = _round_up(B, TM)

    xp = x if Bp == B else jnp.pad(x, ((0, Bp - B), (0, 0)))
    w1p = w1 if Dp == D else jnp.pad(w1, ((0, 0), (0, Dp - D)))
    w2p = w2 if Dp == D else jnp.pad(w2, ((0, Dp - D), (0, Dp - D)))
    b1p = (b1 if Dp == D else jnp.pad(b1, (0, Dp - D))).reshape(1, Dp)
    b2p = (b2 if Dp == D else jnp.pad(b2, (0, Dp - D))).reshape(1, Dp)

    out = pl.pallas_call(
        _mlp_kernel,
        out_shape=jax.ShapeDtypeStruct((Bp, Dp), x.dtype),
        grid=(Bp // TM,),
        in_specs=[
            pl.BlockSpec((TM, Din), lambda i: (i, 0)),
            pl.BlockSpec((Din, Dp), lambda i: (0, 0)),
            pl.BlockSpec((1, Dp), lambda i: (0, 0)),
            pl.BlockSpec((Dp, Dp), lambda i: (0, 0)),
            pl.BlockSpec((1, Dp), lambda i: (0, 0)),
        ],
        out_specs=pl.BlockSpec((TM, Dp), lambda i: (i, 0)),
        scratch_shapes=[
            pltpu.VMEM((Din, Dp), jnp.bfloat16),
            pltpu.VMEM((Dp, Dp), jnp.bfloat16),
        ],
        compiler_params=pltpu.CompilerParams(
            dimension_semantics=("arbitrary",),
            vmem_limit_bytes=48 * 1024 * 1024,
        ),
    )(xp, w1p, b1p, w2p, b2p)

    return out[:B, :D]


# speedup vs baseline: 1.0413x; 1.0263x over previous
import jax
import jax.numpy as jnp
from jax.experimental import pallas as pl
from jax.experimental.pallas import tpu as pltpu


def _round_up(v, m):
    return ((v + m - 1) // m) * m


def _mlp_kernel(x_ref, w1_ref, b1_ref, w2_ref, b2_ref, o_ref):
    tm = x_ref.shape[0]
    sub = 512 if tm % 512 == 0 else tm
    for j in range(tm // sub):
        rows = pl.ds(j * sub, sub)
        h = jnp.dot(x_ref[rows, :], w1_ref[...],
                    preferred_element_type=jnp.float32)
        h = h + b1_ref[...]
        h = h * jax.nn.sigmoid(h)
        out = jnp.dot(h, w2_ref[...], preferred_element_type=jnp.float32)
        o_ref[rows, :] = (out + b2_ref[...]).astype(o_ref.dtype)


def kernel(w1, b1, w2, b2, x):
    B, Din = x.shape
    D = w1.shape[1]

    Dp = _round_up(D, 128)
    TM = 2048 if B % 4096 == 0 else _round_up(min(512, B), 8)
    Bp = _round_up(B, TM)

    xp = x if Bp == B else jnp.pad(x, ((0, Bp - B), (0, 0)))
    w1p = w1 if Dp == D else jnp.pad(w1, ((0, 0), (0, Dp - D)))
    w2p = w2 if Dp == D else jnp.pad(w2, ((0, Dp - D), (0, Dp - D)))
    b1p = (b1 if Dp == D else jnp.pad(b1, (0, Dp - D))).reshape(1, Dp)
    b2p = (b2 if Dp == D else jnp.pad(b2, (0, Dp - D))).reshape(1, Dp)

    out = pl.pallas_call(
        _mlp_kernel,
        out_shape=jax.ShapeDtypeStruct((Bp, Dp), x.dtype),
        grid=(Bp // TM,),
        in_specs=[
            pl.BlockSpec((TM, Din), lambda i: (i, 0)),
            pl.BlockSpec((Din, Dp), lambda i: (0, 0)),
            pl.BlockSpec((1, Dp), lambda i: (0, 0)),
            pl.BlockSpec((Dp, Dp), lambda i: (0, 0)),
            pl.BlockSpec((1, Dp), lambda i: (0, 0)),
        ],
        out_specs=pl.BlockSpec((TM, Dp), lambda i: (i, 0)),
        compiler_params=pltpu.CompilerParams(
            dimension_semantics=("parallel",),
            vmem_limit_bytes=60 * 1024 * 1024,
        ),
    )(xp, w1p, b1p, w2p, b2p)

    return out[:B, :D]
